# baseline (speedup 1.0000x reference)
import functools

import jax
import jax.numpy as jnp
from jax import lax
from jax.experimental import pallas as pl
from jax.experimental.pallas import tpu as pltpu
from jax.experimental.pallas import tpu_sc as plsc

B = 16
CT = 32
TT = 32
R = 31
TILE = 8 * 128
SLAB = TT * TILE
OSLAB = 2 * TILE
NW = 32
SPW = (B * CT) // NW


def _sc_chunk(x_lin):
    mesh = plsc.VectorSubcoreMesh(core_axis_name="c", subcore_axis_name="s")

    @functools.partial(
        pl.kernel,
        out_type=jax.ShapeDtypeStruct((B * R * CT * OSLAB,), jnp.float32),
        mesh=mesh,
        compiler_params=pltpu.CompilerParams(
            needs_layout_passes=False,
            disable_bounds_checks=True,
            disable_semaphore_checks=True,
        ),
        scratch_types=[
            pltpu.VMEM((SLAB,), jnp.float32),
            pltpu.VMEM((SLAB,), jnp.float32),
            pltpu.SemaphoreType.DMA,
            pltpu.SemaphoreType.DMA,
            pltpu.SemaphoreType.DMA,
            pltpu.SemaphoreType.DMA,
        ],
    )
    def k(x_hbm, out_hbm, buf0, buf1, si0, si1, so0, so1):
        wid = lax.axis_index("s") * 2 + lax.axis_index("c")
        bufs, sis, sos = (buf0, buf1), (si0, si1), (so0, so1)

        def in_dma(i, p):
            s = wid + i * NW
            return pltpu.make_async_copy(
                x_hbm.at[pl.ds(s * SLAB, SLAB)], bufs[p], sis[p])

        def out_dma(i, r, p):
            s = wid + i * NW
            b, ct = s >> 5, s & 31
            off = ((b * R + r) * CT + ct) * OSLAB
            return pltpu.make_async_copy(
                bufs[p].at[pl.ds(r * TILE, OSLAB)],
                out_hbm.at[pl.ds(off, OSLAB)], sos[p])

        def outs_start(i, p):
            def body(r, _):
                out_dma(i, r, p).start()
                return 0

            lax.fori_loop(0, R, body, 0)

        def outs_wait(i, p):
            def body(r, _):
                out_dma(i, r, p).wait()
                return 0

            lax.fori_loop(0, R, body, 0)

        in_dma(0, 0).start()

        def step(i, p):
            in_dma(i, p).wait()
            outs_start(i, p)

            @pl.when(i + 1 < SPW)
            def _():
                @pl.when(i >= 1)
                def _():
                    outs_wait(i - 1, 1 - p)

                in_dma(i + 1, 1 - p).start()

        def pair(k2, _):
            step(k2 * 2, 0)
            step(k2 * 2 + 1, 1)
            return 0

        lax.fori_loop(0, SPW // 2, pair, 0)
        outs_wait(SPW - 2, 0)
        outs_wait(SPW - 1, 1)

    return k(x_lin)


def kernel(x):
    x_lin = x.reshape(B, CT, 8, TT, 128).transpose(0, 1, 3, 2, 4).reshape(-1)
    out_lin = _sc_chunk(x_lin)
    out = (out_lin.reshape(B, R, CT, 2, 8, 128)
           .transpose(0, 2, 4, 3, 5, 1)
           .reshape(16, 256, 256, 31))
    return out


# speedup vs baseline: 1.0049x; 1.0049x over previous
import functools

import jax
import jax.numpy as jnp
from jax import lax
from jax.experimental import pallas as pl
from jax.experimental.pallas import tpu as pltpu
from jax.experimental.pallas import tpu_sc as plsc

B = 16
CT = 32
TT = 32
R = 31
TILE = 8 * 128
SLAB = TT * TILE
OSLAB = 2 * TILE
NW = 32
SPW = (B * CT) // NW


def _sc_chunk(x_lin):
    mesh = plsc.VectorSubcoreMesh(core_axis_name="c", subcore_axis_name="s")

    @functools.partial(
        pl.kernel,
        out_type=jax.ShapeDtypeStruct((B * R * CT * OSLAB,), jnp.float32),
        mesh=mesh,
        compiler_params=pltpu.CompilerParams(
            needs_layout_passes=False,
            disable_bounds_checks=True,
            disable_semaphore_checks=True,
        ),
        scratch_types=[
            pltpu.VMEM((SLAB,), jnp.float32),
            pltpu.VMEM((SLAB,), jnp.float32),
            pltpu.SemaphoreType.DMA,
            pltpu.SemaphoreType.DMA,
            pltpu.SemaphoreType.DMA,
            pltpu.SemaphoreType.DMA,
        ],
    )
    def k(x_hbm, out_hbm, buf0, buf1, si0, si1, so0, so1):
        wid = lax.axis_index("s") * 2 + lax.axis_index("c")
        bufs, sis, sos = (buf0, buf1), (si0, si1), (so0, so1)

        def in_dma(i, p):
            s = wid + i * NW
            return pltpu.make_async_copy(
                x_hbm.at[pl.ds(s * SLAB, SLAB)], bufs[p], sis[p])

        def out_dma(i, r, p):
            s = wid + i * NW
            b, ct = s >> 5, s & 31
            off = ((b * R + r) * CT + ct) * OSLAB
            return pltpu.make_async_copy(
                bufs[p].at[pl.ds(r * TILE, OSLAB)],
                out_hbm.at[pl.ds(off, OSLAB)], sos[p])

        def outs_start(i, p, lo=0, hi=R):
            def body(r, _):
                out_dma(i, r, p).start()
                return 0

            lax.fori_loop(lo, hi, body, 0)

        def outs_wait(i, p):
            def body(r, _):
                out_dma(i, r, p).wait()
                return 0

            lax.fori_loop(0, R, body, 0)

        QT = TT // 4

        def piece(q):
            return pltpu.make_async_copy(
                x_hbm.at[pl.ds(wid * SLAB + q * QT * TILE, QT * TILE)],
                bufs[0].at[pl.ds(q * QT * TILE, QT * TILE)], sis[0])

        for q in range(4):
            piece(q).start()
        for q in range(4):
            piece(q).wait()
            outs_start(0, 0, max(0, q * QT - 1), (q + 1) * QT - 1)
        in_dma(1, 1).start()

        def step(i, p):
            in_dma(i, p).wait()
            outs_start(i, p)

            @pl.when(i + 1 < SPW)
            def _():
                @pl.when(i >= 1)
                def _():
                    outs_wait(i - 1, 1 - p)

                in_dma(i + 1, 1 - p).start()

        step(1, 1)

        def pair(k2, _):
            step(k2 * 2, 0)
            step(k2 * 2 + 1, 1)
            return 0

        lax.fori_loop(1, SPW // 2, pair, 0)
        outs_wait(SPW - 2, 0)
        outs_wait(SPW - 1, 1)

    return k(x_lin)


def kernel(x):
    x_lin = x.reshape(B, CT, 8, TT, 128).transpose(0, 1, 3, 2, 4).reshape(-1)
    out_lin = _sc_chunk(x_lin)
    out = (out_lin.reshape(B, R, CT, 2, 8, 128)
           .transpose(0, 2, 4, 3, 5, 1)
           .reshape(16, 256, 256, 31))
    return out
